# baseline (speedup 1.0000x reference)
import functools

import jax
import jax.numpy as jnp
from jax import lax
from jax.experimental import pallas as pl
from jax.experimental.pallas import tpu as pltpu
from jax.experimental.pallas import tpu_sc as plsc

N = 10000
E = 160000
D = 256
NPAD = 10240
HALF = D // 2
EPAD = 163840

K2_CH = 64
K2_PER_TILE = 80
K4_CH = 80
K4_PER_TILE = 128


def _k1_body(x_ref, wt_ref, wb_ref, b1_ref, a_ref, b_ref):
    xb = x_ref[...]
    a_ref[...] = (
        jnp.dot(xb, wt_ref[...], preferred_element_type=jnp.float32)
        + b1_ref[0:1, :]
    )
    b_ref[...] = jnp.dot(xb, wb_ref[...], preferred_element_type=jnp.float32)


def _node_tables(xp, W1, b1):
    b1_t = jnp.broadcast_to(b1.reshape(1, D), (8, D))
    return pl.pallas_call(
        _k1_body,
        grid=(NPAD // 256,),
        in_specs=[
            pl.BlockSpec((256, D), lambda i: (i, 0)),
            pl.BlockSpec((D, D), lambda i: (0, 0)),
            pl.BlockSpec((D, D), lambda i: (0, 0)),
            pl.BlockSpec((8, D), lambda i: (0, 0)),
        ],
        out_specs=[
            pl.BlockSpec((256, D), lambda i: (i, 0)),
            pl.BlockSpec((256, D), lambda i: (i, 0)),
        ],
        out_shape=[
            jax.ShapeDtypeStruct((NPAD, D), jnp.float32),
            jax.ShapeDtypeStruct((NPAD, D), jnp.float32),
        ],
    )(xp, W1[:D], W1[D:], b1_t)


def _k2_logits(A, B, pk2, w2):
    mesh = plsc.VectorSubcoreMesh(core_axis_name="c", subcore_axis_name="s")

    @functools.partial(
        pl.kernel,
        out_type=jax.ShapeDtypeStruct((EPAD,), jnp.float32),
        mesh=mesh,
        scratch_types=[
            pltpu.VMEM((2, 2, K2_CH), jnp.int32),
            pltpu.VMEM((2, K2_CH, D), jnp.float32),
            pltpu.VMEM((2, K2_CH, D), jnp.float32),
            pltpu.VMEM((2, K2_CH), jnp.float32),
            pltpu.VMEM((D,), jnp.float32),
            pltpu.SemaphoreType.DMA,
            pltpu.SemaphoreType.DMA,
            pltpu.SemaphoreType.DMA,
            pltpu.SemaphoreType.DMA,
        ],
    )
    def k2(a_hbm, b_hbm, pk_hbm, w2_hbm, out_hbm,
           pkb, arows, brows, lg, w2v, semA0, semA1, semB0, semB1):
        wid = lax.axis_index("s") * 2 + lax.axis_index("c")
        lane = lax.iota(jnp.int32, 16)
        pltpu.sync_copy(w2_hbm, w2v)
        semsA = (semA0, semA1)
        semsB = (semB0, semB1)

        def chunk_of(j):
            return wid * K2_PER_TILE + jnp.minimum(j, K2_PER_TILE - 1)

        def off_of(j):
            return pl.multiple_of(chunk_of(j) * K2_CH, K2_CH)

        def fire(j, b):
            pltpu.sync_copy(pk_hbm.at[chunk_of(j)], pkb.at[b])
            pltpu.async_copy(a_hbm.at[pkb.at[b, 0]], arows.at[b], semsA[b])
            pltpu.async_copy(b_hbm.at[pkb.at[b, 1]], brows.at[b], semsB[b])

        def wait(b):
            pltpu.make_async_copy(a_hbm.at[pkb.at[b, 0]], arows.at[b], semsA[b]).wait()
            pltpu.make_async_copy(b_hbm.at[pkb.at[b, 1]], brows.at[b], semsB[b]).wait()

        def lane_shuffle(v, idx):
            return lax.gather(
                v, idx[:, None],
                dimension_numbers=lax.GatherDimensionNumbers(
                    offset_dims=(), collapsed_slice_dims=(0,),
                    start_index_map=(0,)),
                slice_sizes=(1,),
                mode=lax.GatherScatterMode.PROMISE_IN_BOUNDS)

        def compute(j, b):
            ar = arows.at[b]
            br = brows.at[b]
            lgb = lg.at[b]
            w2r = [w2v[pl.ds(k * 16, 16)] for k in range(D // 16)]
            for g in range(K2_CH // 16):
                def ebody(jj, lgvec):
                    e = g * 16 + jj
                    acc0 = jnp.zeros((16,), jnp.float32)
                    acc1 = jnp.zeros((16,), jnp.float32)
                    for k in range(0, D // 16, 2):
                        av0 = ar[e, pl.ds(k * 16, 16)]
                        bv0 = br[e, pl.ds(k * 16, 16)]
                        av1 = ar[e, pl.ds((k + 1) * 16, 16)]
                        bv1 = br[e, pl.ds((k + 1) * 16, 16)]
                        acc0 = acc0 + jnp.maximum(av0 + bv0, 0.0) * w2r[k]
                        acc1 = acc1 + jnp.maximum(av1 + bv1, 0.0) * w2r[k + 1]
                    acc = acc0 + acc1
                    for s in (8, 4, 2, 1):
                        acc = acc + lane_shuffle(acc, lane ^ s)
                    return jnp.where(lane == jj, acc, lgvec)

                lgvec = lax.fori_loop(0, 16, ebody, jnp.zeros((16,), jnp.float32))
                lgb[pl.ds(g * 16, 16)] = lgvec
            pltpu.sync_copy(lgb, out_hbm.at[pl.ds(off_of(j), K2_CH)])

        fire(0, 0)
        fire(1, 1)

        def pair(jj, carry):
            for b in (0, 1):
                j = jj * 2 + b
                wait(b)
                compute(j, b)
                fire(j + 2, b)
            return carry

        lax.fori_loop(0, K2_PER_TILE // 2, pair, 0)
        wait(0)
        wait(1)

    return k2(A, B, pk2, w2)


def _k3_body(lg_ref, w_ref):
    v = lg_ref[...]
    rows = lax.broadcasted_iota(jnp.int32, v.shape, 0)
    lanes = lax.broadcasted_iota(jnp.int32, v.shape, 1)
    idx = rows * v.shape[1] + lanes
    v = jnp.where(idx < E, v, -1e30)
    m = jnp.max(v)
    p = jnp.exp(v - m)
    s = jnp.sum(p)
    w_ref[...] = p / s


def _softmax(logits_pad_2d):
    return pl.pallas_call(
        _k3_body,
        out_shape=jax.ShapeDtypeStruct(logits_pad_2d.shape, jnp.float32),
    )(logits_pad_2d)


def _k4_scatter(xV, pk, zrows):
    mesh = plsc.VectorSubcoreMesh(core_axis_name="c", subcore_axis_name="s")

    @functools.partial(
        pl.kernel,
        out_type=jax.ShapeDtypeStruct((NPAD, D), jnp.float32),
        mesh=mesh,
        scratch_types=[
            pltpu.VMEM((2, 3, K4_CH), jnp.int32),
            pltpu.VMEM((2, K4_CH, HALF), jnp.float32),
            pltpu.VMEM((2, K4_CH), jnp.int32),
            pltpu.VMEM((2, K4_CH, HALF), jnp.float32),
            pltpu.VMEM_SHARED((NPAD, HALF), jnp.float32),
            pltpu.SemaphoreType.DMA,
            pltpu.SemaphoreType.DMA,
            pltpu.SemaphoreType.DMA,
            pltpu.SemaphoreType.DMA,
            pltpu.SemaphoreType.DMA,
            pltpu.SemaphoreType.DMA,
        ],
    )
    def k4(x_hbm, pk_hbm, z_hbm, out_hbm,
           pkb, xrows, sidx, sbuf, acc,
           semG0, semG1, semS0, semS1, semI0, semI1):
        cid = lax.axis_index("c")
        tid = lax.axis_index("s")
        gsems = (semG0, semG1)
        ssems = (semS0, semS1)
        isems = (semI0, semI1)

        pltpu.sync_copy(z_hbm, acc.at[pl.ds(tid * (NPAD // 16), NPAD // 16)])
        plsc.subcore_barrier()

        rbias = cid * N

        def chunk_of(j):
            return tid * K4_PER_TILE + jnp.minimum(j, K4_PER_TILE - 1)

        def fire_idx(j, q):
            pltpu.async_copy(pk_hbm.at[chunk_of(j)], pkb.at[q], isems[q])

        def wait_idx(q):
            pltpu.make_async_copy(
                pk_hbm.at[chunk_of(0)], pkb.at[q], isems[q]).wait()

        def fire_gather(q, b):
            rb = pkb.at[q, 0]
            for g in range(K4_CH // 16):
                rb[pl.ds(g * 16, 16)] = rb[pl.ds(g * 16, 16)] + rbias
            pltpu.async_copy(x_hbm.at[rb], xrows.at[b], gsems[b])

        def wait_gather(q, b):
            pltpu.make_async_copy(
                x_hbm.at[pkb.at[q, 0]], xrows.at[b], gsems[b]).wait()

        def wait_scatter(b):
            pltpu.make_async_copy(
                sbuf.at[b], acc.at[sidx.at[b]], ssems[b]).wait()

        def bcast(v, j):
            idx = jnp.zeros((16,), jnp.int32) + j
            return lax.gather(
                v, idx[:, None],
                dimension_numbers=lax.GatherDimensionNumbers(
                    offset_dims=(), collapsed_slice_dims=(0,),
                    start_index_map=(0,)),
                slice_sizes=(1,),
                mode=lax.GatherScatterMode.PROMISE_IN_BOUNDS)

        def step(jj, j, b):
            wait_gather(b, b)
            @pl.when(jj > 0)
            def _():
                wait_scatter(b)

            xr = xrows.at[b]
            sb = sbuf.at[b]
            si = sidx.at[b]
            cb = pkb.at[b, 1]
            wb = pkb.at[b, 2]
            w16s = []
            for g in range(K4_CH // 16):
                si[pl.ds(g * 16, 16)] = cb[pl.ds(g * 16, 16)]
                w16s.append(
                    wb[pl.ds(g * 16, 16)].astype(jnp.float32) * (1.0 / (1 << 30)))
            wait_idx(1 - b)
            fire_gather(1 - b, 1 - b)
            fire_idx(j + 2, b)

            for g in range(K4_CH // 16):
                for inner in range(16):
                    e = g * 16 + inner
                    wbc = bcast(w16s[g], inner)
                    for k in range(HALF // 16):
                        sb[e, pl.ds(k * 16, 16)] = xr[e, pl.ds(k * 16, 16)] * wbc
            pltpu.async_copy(sb, acc.at[si], ssems[b], add=True)

        fire_idx(0, 0)
        fire_idx(1, 1)
        wait_idx(0)
        fire_gather(0, 0)

        def pair(jj, carry):
            for b in (0, 1):
                step(jj, jj * 2 + b, b)
            return carry

        lax.fori_loop(0, K4_PER_TILE // 2, pair, 0)
        wait_gather(0, 0)
        wait_idx(1)
        wait_scatter(0)
        wait_scatter(1)

        plsc.subcore_barrier()
        pltpu.sync_copy(
            acc.at[pl.ds(tid * (NPAD // 16), NPAD // 16)],
            out_hbm.at[pl.ds(tid * (NPAD // 16), NPAD // 16),
                       pl.ds(cid * HALF, HALF)],
        )

    return k4(xV, pk, zrows)


def kernel(x, edge_index, W1, b1, W2, b2):
    row = edge_index[0]
    col = edge_index[1]

    npad_e = EPAD - E
    rowp = jnp.concatenate(
        [row, (jnp.arange(npad_e, dtype=jnp.int32) * 61) % N])
    colp = jnp.concatenate(
        [col, N + (jnp.arange(npad_e, dtype=jnp.int32) % (NPAD - N))])

    A, B = _node_tables(x, W1, b1)

    nc2 = EPAD // K2_CH
    pk2 = jnp.stack(
        [rowp.reshape(nc2, K2_CH), colp.reshape(nc2, K2_CH)], axis=1)
    logits = _k2_logits(A, B, pk2, W2.reshape(D))
    w2d = _softmax(logits.reshape(EPAD // 128, 128))
    wflat = w2d.reshape(-1)

    xV = jnp.concatenate([x[:, :HALF], x[:, HALF:]], axis=0)
    zrows = jnp.zeros((NPAD // 16, HALF), jnp.float32)
    nc4 = EPAD // K4_CH
    wq = (wflat * float(1 << 30)).astype(jnp.int32)
    pk = jnp.stack(
        [rowp.reshape(nc4, K4_CH), colp.reshape(nc4, K4_CH),
         wq.reshape(nc4, K4_CH)], axis=1)
    outp = _k4_scatter(xV, pk, zrows)
    attention_weights = wflat[:E].reshape(E, 1)
    return outp[:N], attention_weights


# speedup vs baseline: 1.0042x; 1.0042x over previous
import functools

import jax
import jax.numpy as jnp
from jax import lax
from jax.experimental import pallas as pl
from jax.experimental.pallas import tpu as pltpu
from jax.experimental.pallas import tpu_sc as plsc

N = 10000
E = 160000
D = 256
NPAD = 10240
HALF = D // 2
EPAD = 163840

K2_CH = 64
K2_PER_TILE = 80
K4_CH = 80
K4_PER_TILE = 128


def _k1_body(x_ref, wt_ref, wb_ref, b1_ref, a_ref, b_ref):
    xb = x_ref[...]
    a_ref[...] = (
        jnp.dot(xb, wt_ref[...], preferred_element_type=jnp.float32)
        + b1_ref[0:1, :]
    )
    b_ref[...] = jnp.dot(xb, wb_ref[...], preferred_element_type=jnp.float32)


def _node_tables(xp, W1, b1):
    b1_t = jnp.broadcast_to(b1.reshape(1, D), (8, D))
    return pl.pallas_call(
        _k1_body,
        grid=(NPAD // 256,),
        in_specs=[
            pl.BlockSpec((256, D), lambda i: (i, 0)),
            pl.BlockSpec((D, D), lambda i: (0, 0)),
            pl.BlockSpec((D, D), lambda i: (0, 0)),
            pl.BlockSpec((8, D), lambda i: (0, 0)),
        ],
        out_specs=[
            pl.BlockSpec((256, D), lambda i: (i, 0)),
            pl.BlockSpec((256, D), lambda i: (i, 0)),
        ],
        out_shape=[
            jax.ShapeDtypeStruct((NPAD, D), jnp.float32),
            jax.ShapeDtypeStruct((NPAD, D), jnp.float32),
        ],
    )(xp, W1[:D], W1[D:], b1_t)


def _k2_logits(A, B, pk2, w2):
    mesh = plsc.VectorSubcoreMesh(core_axis_name="c", subcore_axis_name="s")

    @functools.partial(
        pl.kernel,
        out_type=jax.ShapeDtypeStruct((EPAD,), jnp.float32),
        mesh=mesh,
        scratch_types=[
            pltpu.VMEM((2, 2, K2_CH), jnp.int32),
            pltpu.VMEM((2, K2_CH, D), jnp.float32),
            pltpu.VMEM((2, K2_CH, D), jnp.float32),
            pltpu.VMEM((2, K2_CH), jnp.float32),
            pltpu.VMEM((D,), jnp.float32),
            pltpu.SemaphoreType.DMA,
            pltpu.SemaphoreType.DMA,
            pltpu.SemaphoreType.DMA,
            pltpu.SemaphoreType.DMA,
        ],
    )
    def k2(a_hbm, b_hbm, pk_hbm, w2_hbm, out_hbm,
           pkb, arows, brows, lg, w2v, semA0, semA1, semB0, semB1):
        wid = lax.axis_index("s") * 2 + lax.axis_index("c")
        lane = lax.iota(jnp.int32, 16)
        pltpu.sync_copy(w2_hbm, w2v)
        semsA = (semA0, semA1)
        semsB = (semB0, semB1)

        def chunk_of(j):
            return wid * K2_PER_TILE + jnp.minimum(j, K2_PER_TILE - 1)

        def off_of(j):
            return pl.multiple_of(chunk_of(j) * K2_CH, K2_CH)

        def fire(j, b):
            pltpu.sync_copy(pk_hbm.at[chunk_of(j)], pkb.at[b])
            pltpu.async_copy(a_hbm.at[pkb.at[b, 0]], arows.at[b], semsA[b])
            pltpu.async_copy(b_hbm.at[pkb.at[b, 1]], brows.at[b], semsB[b])

        def wait(b):
            pltpu.make_async_copy(a_hbm.at[pkb.at[b, 0]], arows.at[b], semsA[b]).wait()
            pltpu.make_async_copy(b_hbm.at[pkb.at[b, 1]], brows.at[b], semsB[b]).wait()

        def lane_shuffle(v, idx):
            return lax.gather(
                v, idx[:, None],
                dimension_numbers=lax.GatherDimensionNumbers(
                    offset_dims=(), collapsed_slice_dims=(0,),
                    start_index_map=(0,)),
                slice_sizes=(1,),
                mode=lax.GatherScatterMode.PROMISE_IN_BOUNDS)

        def compute(j, b):
            ar = arows.at[b]
            br = brows.at[b]
            lgb = lg.at[b]
            w2r = [w2v[pl.ds(k * 16, 16)] for k in range(D // 16)]
            for g in range(K2_CH // 16):
                def ebody(jj, lgvec):
                    e = g * 16 + jj
                    acc = jnp.zeros((16,), jnp.float32)
                    for k in range(D // 16):
                        av = ar[e, pl.ds(k * 16, 16)]
                        bv = br[e, pl.ds(k * 16, 16)]
                        acc = acc + jnp.maximum(av + bv, 0.0) * w2r[k]
                    for s in (8, 4, 2, 1):
                        acc = acc + lane_shuffle(acc, lane ^ s)
                    return jnp.where(lane == jj, acc, lgvec)

                lgvec = lax.fori_loop(0, 16, ebody, jnp.zeros((16,), jnp.float32))
                lgb[pl.ds(g * 16, 16)] = lgvec
            pltpu.sync_copy(lgb, out_hbm.at[pl.ds(off_of(j), K2_CH)])

        fire(0, 0)
        fire(1, 1)

        def pair(jj, carry):
            for b in (0, 1):
                j = jj * 2 + b
                wait(b)
                compute(j, b)
                fire(j + 2, b)
            return carry

        lax.fori_loop(0, K2_PER_TILE // 2, pair, 0)
        wait(0)
        wait(1)

    return k2(A, B, pk2, w2)


def _k3_body(lg_ref, w_ref):
    v = lg_ref[...]
    rows = lax.broadcasted_iota(jnp.int32, v.shape, 0)
    lanes = lax.broadcasted_iota(jnp.int32, v.shape, 1)
    idx = rows * v.shape[1] + lanes
    v = jnp.where(idx < E, v, -1e30)
    m = jnp.max(v)
    p = jnp.exp(v - m)
    s = jnp.sum(p)
    w_ref[...] = p / s


def _softmax(logits_pad_2d):
    return pl.pallas_call(
        _k3_body,
        out_shape=jax.ShapeDtypeStruct(logits_pad_2d.shape, jnp.float32),
    )(logits_pad_2d)


def _k4_scatter(xV, pk, zrows):
    mesh = plsc.VectorSubcoreMesh(core_axis_name="c", subcore_axis_name="s")

    @functools.partial(
        pl.kernel,
        out_type=jax.ShapeDtypeStruct((NPAD, D), jnp.float32),
        mesh=mesh,
        scratch_types=[
            pltpu.VMEM((2, 3, K4_CH), jnp.int32),
            pltpu.VMEM((2, K4_CH, HALF), jnp.float32),
            pltpu.VMEM((2, K4_CH), jnp.int32),
            pltpu.VMEM((2, K4_CH, HALF), jnp.float32),
            pltpu.VMEM_SHARED((NPAD, HALF), jnp.float32),
            pltpu.SemaphoreType.DMA,
            pltpu.SemaphoreType.DMA,
            pltpu.SemaphoreType.DMA,
            pltpu.SemaphoreType.DMA,
            pltpu.SemaphoreType.DMA,
            pltpu.SemaphoreType.DMA,
        ],
    )
    def k4(x_hbm, pk_hbm, z_hbm, out_hbm,
           pkb, xrows, sidx, sbuf, acc,
           semG0, semG1, semS0, semS1, semI0, semI1):
        cid = lax.axis_index("c")
        tid = lax.axis_index("s")
        gsems = (semG0, semG1)
        ssems = (semS0, semS1)
        isems = (semI0, semI1)

        pltpu.sync_copy(z_hbm, acc.at[pl.ds(tid * (NPAD // 16), NPAD // 16)])
        plsc.subcore_barrier()

        rbias = cid * N

        def chunk_of(j):
            return tid * K4_PER_TILE + jnp.minimum(j, K4_PER_TILE - 1)

        def fire_idx(j, q):
            pltpu.async_copy(pk_hbm.at[chunk_of(j)], pkb.at[q], isems[q])

        def wait_idx(q):
            pltpu.make_async_copy(
                pk_hbm.at[chunk_of(0)], pkb.at[q], isems[q]).wait()

        def fire_gather(q, b):
            rb = pkb.at[q, 0]
            for g in range(K4_CH // 16):
                rb[pl.ds(g * 16, 16)] = rb[pl.ds(g * 16, 16)] + rbias
            pltpu.async_copy(x_hbm.at[rb], xrows.at[b], gsems[b])

        def wait_gather(q, b):
            pltpu.make_async_copy(
                x_hbm.at[pkb.at[q, 0]], xrows.at[b], gsems[b]).wait()

        def wait_scatter(b):
            pltpu.make_async_copy(
                sbuf.at[b], acc.at[sidx.at[b]], ssems[b]).wait()

        def bcast(v, j):
            idx = jnp.zeros((16,), jnp.int32) + j
            return lax.gather(
                v, idx[:, None],
                dimension_numbers=lax.GatherDimensionNumbers(
                    offset_dims=(), collapsed_slice_dims=(0,),
                    start_index_map=(0,)),
                slice_sizes=(1,),
                mode=lax.GatherScatterMode.PROMISE_IN_BOUNDS)

        def step(jj, j, b):
            wait_gather(b, b)
            @pl.when(jj > 0)
            def _():
                wait_scatter(b)

            xr = xrows.at[b]
            sb = sbuf.at[b]
            si = sidx.at[b]
            cb = pkb.at[b, 1]
            wb = pkb.at[b, 2]
            w16s = []
            for g in range(K4_CH // 16):
                si[pl.ds(g * 16, 16)] = cb[pl.ds(g * 16, 16)]
                w16s.append(
                    wb[pl.ds(g * 16, 16)].astype(jnp.float32) * (1.0 / (1 << 30)))
            wait_idx(1 - b)
            fire_gather(1 - b, 1 - b)
            fire_idx(j + 2, b)

            for g in range(K4_CH // 16):
                for inner in range(16):
                    e = g * 16 + inner
                    wbc = bcast(w16s[g], inner)
                    for k in range(HALF // 16):
                        sb[e, pl.ds(k * 16, 16)] = xr[e, pl.ds(k * 16, 16)] * wbc
            pltpu.async_copy(sb, acc.at[si], ssems[b], add=True)

        fire_idx(0, 0)
        fire_idx(1, 1)
        wait_idx(0)
        fire_gather(0, 0)

        def pair(jj, carry):
            for b in (0, 1):
                step(jj, jj * 2 + b, b)
            return carry

        lax.fori_loop(0, K4_PER_TILE // 2, pair, 0)
        wait_gather(0, 0)
        wait_idx(1)
        wait_scatter(0)
        wait_scatter(1)

        plsc.subcore_barrier()
        pltpu.sync_copy(
            acc.at[pl.ds(tid * (NPAD // 16), NPAD // 16)],
            out_hbm.at[pl.ds(tid * (NPAD // 16), NPAD // 16),
                       pl.ds(cid * HALF, HALF)],
        )

    return k4(xV, pk, zrows)


def kernel(x, edge_index, W1, b1, W2, b2):
    row = edge_index[0]
    col = edge_index[1]

    npad_e = EPAD - E
    rowp = jnp.concatenate(
        [row, (jnp.arange(npad_e, dtype=jnp.int32) * 61) % N])
    colp = jnp.concatenate(
        [col, N + (jnp.arange(npad_e, dtype=jnp.int32) % (NPAD - N))])

    A, B = _node_tables(x, W1, b1)

    nc2 = EPAD // K2_CH
    pk2 = jnp.stack(
        [rowp.reshape(nc2, K2_CH), colp.reshape(nc2, K2_CH)], axis=1)
    logits = _k2_logits(A, B, pk2, W2.reshape(D))
    w2d = _softmax(logits.reshape(EPAD // 128, 128))
    wflat = w2d.reshape(-1)

    xV = jnp.concatenate([x[:, :HALF], x[:, HALF:]], axis=0)
    zrows = jnp.zeros((NPAD // 16, HALF), jnp.float32)
    nc4 = EPAD // K4_CH
    wq = (wflat * float(1 << 30)).astype(jnp.int32)
    pk = jnp.stack(
        [rowp.reshape(nc4, K4_CH), colp.reshape(nc4, K4_CH),
         wq.reshape(nc4, K4_CH)], axis=1)
    outp = _k4_scatter(xV, pk, zrows)
    attention_weights = wflat[:E].reshape(E, 1)
    return outp[:N], attention_weights
